# baseline (speedup 1.0000x reference)
import jax
import jax.numpy as jnp
from jax import lax
from jax.experimental import pallas as pl
from jax.experimental.pallas import tpu as pltpu
from jax.experimental.pallas import tpu_sc as plsc

D = 128
G = 8
NBUF = 2

_info = plsc.get_sparse_core_info()
_NC, _NS = _info.num_cores, _info.num_subcores
NW = _NC * _NS


def _body(x_hbm, table_hbm, out_hbm, idx_v, *scratch):
    wid = lax.axis_index("s") * _NC + lax.axis_index("c")
    rows_pw = x_hbm.shape[0] // NW
    ngroup = rows_pw // G
    base = wid * rows_pw
    rows = scratch[:NBUF]
    gsem = scratch[NBUF:2 * NBUF]
    wsem = scratch[2 * NBUF:]

    pltpu.sync_copy(x_hbm.at[pl.ds(base, rows_pw)], idx_v)

    @pl.loop(0, ngroup, step=NBUF)
    def _(t0):
        for b in range(NBUF):
            t = t0 + b

            @pl.when(t0 >= NBUF)
            def _():
                pltpu.make_async_copy(
                    rows[b],
                    out_hbm.at[pl.ds(base + (t - NBUF) * G, G)],
                    wsem[b],
                ).wait()

            for g in range(G):
                pltpu.async_copy(
                    table_hbm.at[idx_v.at[t * G + g]],
                    rows[b].at[g],
                    gsem[b],
                )

        for b in range(NBUF):
            t = t0 + b
            for g in range(G):
                pltpu.make_async_copy(
                    table_hbm.at[idx_v.at[t * G + g]], rows[b].at[g], gsem[b]
                ).wait()
            pltpu.async_copy(
                rows[b], out_hbm.at[pl.ds(base + t * G, G)], wsem[b]
            )

    for b in range(NBUF):
        t = ngroup - NBUF + b
        pltpu.make_async_copy(
            rows[b], out_hbm.at[pl.ds(base + t * G, G)], wsem[b]
        ).wait()


@jax.jit
def kernel(x, table):
    batch, hist = x.shape
    rows_pw = batch // NW

    mesh = plsc.VectorSubcoreMesh(core_axis_name="c", subcore_axis_name="s")
    run = pl.kernel(
        _body,
        out_type=jax.ShapeDtypeStruct((batch, hist, D), jnp.float32),
        mesh=mesh,
        scratch_types=(
            [pltpu.VMEM((rows_pw, hist), jnp.int32)]
            + [pltpu.VMEM((G, hist, D), jnp.float32)] * NBUF
            + [pltpu.SemaphoreType.DMA] * (2 * NBUF)
        ),
    )
    return run(x.astype(jnp.int32), table)
